# Initial kernel scaffold; baseline (speedup 1.0000x reference)
import jax
import jax.numpy as jnp
from jax.experimental import pallas as pl


def kernel(points, features, W1, b1, W2, b2):
    raise NotImplementedError("write your pallas kernel here")


# speedup vs baseline: 22.8491x; 22.8491x over previous
import jax
import jax.numpy as jnp
from jax import lax
from jax.experimental import pallas as pl

_K = 16


def _gelu(x):
    return 0.5 * x * (1.0 + lax.erf(x * 0.7071067811865476))


def _fused_body(pc_ref, pct_ref, ft_ref, a1_ref, c1_ref, w2t_ref, b1_ref,
                b2_ref, out_ref):
    n = pc_ref.shape[1]
    pcb = pc_ref[0]
    pct = pct_ref[0]
    ftb = ft_ref[0]

    m = jnp.dot(pcb, pct, preferred_element_type=jnp.float32)
    r_row = jnp.sum(pcb * pcb, axis=1, keepdims=True)
    r_col = jnp.sum(pct * pct, axis=0, keepdims=True)
    d = r_row - 2.0 * m + r_col + 1e-5

    cols = lax.broadcasted_iota(jnp.int32, (n, n), 1)
    nbs = []
    for t in range(_K + 1):
        minval = jnp.min(d, axis=1, keepdims=True)
        am = jnp.min(jnp.where(d == minval, cols, n), axis=1, keepdims=True)
        sel = cols == am
        if t > 0:
            mask = jnp.where(sel, 1.0, 0.0).astype(jnp.float32)
            nbs.append(jnp.dot(mask, ftb, preferred_element_type=jnp.float32))
        d = jnp.where(sel, jnp.float32(jnp.inf), d)

    knn = jnp.concatenate(nbs, axis=0)
    c = jnp.dot(ftb, c1_ref[...], preferred_element_type=jnp.float32) + b1_ref[...]
    h1 = jnp.dot(knn, a1_ref[...], preferred_element_type=jnp.float32)
    h1 = _gelu(h1.reshape(_K, n, -1) + c[None])
    h2 = jnp.dot(h1.reshape(_K * n, -1), w2t_ref[...],
                 preferred_element_type=jnp.float32) + b2_ref[...]
    h2 = _gelu(h2)
    out_ref[0] = jnp.mean(h2.reshape(_K, n, -1), axis=0)


def kernel(points, features, W1, b1, W2, b2):
    n, b, pdim = points.shape
    f = features.shape[-1]
    h2dim = W1.shape[0]
    h = W2.shape[0]

    pc = jnp.transpose(points, (1, 0, 2))
    pcp = jnp.pad(pc, ((0, 0), (0, 0), (0, 8 - pdim)))
    pct = jnp.transpose(pcp, (0, 2, 1))
    ft = jnp.transpose(features, (1, 0, 2))
    w1t = W1.T
    a1 = w1t[:f]
    c1 = w1t[f:] - a1
    w2t = W2.T

    out = pl.pallas_call(
        _fused_body,
        grid=(b,),
        in_specs=[
            pl.BlockSpec((1, n, 8), lambda i: (i, 0, 0)),
            pl.BlockSpec((1, 8, n), lambda i: (i, 0, 0)),
            pl.BlockSpec((1, n, f), lambda i: (i, 0, 0)),
            pl.BlockSpec((f, h2dim), lambda i: (0, 0)),
            pl.BlockSpec((f, h2dim), lambda i: (0, 0)),
            pl.BlockSpec((h2dim, h), lambda i: (0, 0)),
            pl.BlockSpec((1, h2dim), lambda i: (0, 0)),
            pl.BlockSpec((1, h), lambda i: (0, 0)),
        ],
        out_specs=pl.BlockSpec((1, n, h), lambda i: (i, 0, 0)),
        out_shape=jax.ShapeDtypeStruct((b, n, h), jnp.float32),
    )(pcp, pct, ft, a1, c1, w2t, b1.reshape(1, -1), b2.reshape(1, -1))
    return jnp.transpose(out, (1, 0, 2))
